# Initial kernel scaffold; baseline (speedup 1.0000x reference)
import functools

import jax
import jax.numpy as jnp
from jax import lax
from jax.experimental import pallas as pl
from jax.experimental.pallas import tpu as pltpu
from jax.experimental.pallas import tpu_sc as plsc

N_NODES = 10000
HIDDEN = 128
N_EDGES = 320000
EDGES_IN_D = 4
NORM = 100.0

NC = 2
NS = 16
NW = NC * NS
CHUNK = 128
NCHUNKS = N_EDGES // CHUNK
CH_PER_TILE = -(-NCHUNKS // NW)
SC_NCHUNKS = NCHUNKS // NC
SC_CH_PER_TILE = -(-SC_NCHUNKS // NS)
ROWS_PER_TILE = N_NODES // NS
PAD = 16

_SC_MESH = plsc.VectorSubcoreMesh(core_axis_name="c", subcore_axis_name="s")


@functools.partial(
    pl.kernel,
    out_type=[
        jax.ShapeDtypeStruct((N_EDGES, HIDDEN), jnp.float32),
        jax.ShapeDtypeStruct((N_EDGES, HIDDEN), jnp.float32),
    ],
    mesh=_SC_MESH,
    scratch_types=[
        pltpu.VMEM((CHUNK,), jnp.int32),
        pltpu.VMEM((CHUNK,), jnp.int32),
        pltpu.VMEM((CHUNK, HIDDEN), jnp.float32),
        pltpu.VMEM((CHUNK, HIDDEN), jnp.float32),
        pltpu.SemaphoreType.DMA,
        pltpu.SemaphoreType.DMA,
    ],
)
def _sc_gather(h_hbm, row_hbm, col_hbm, ga_hbm, gb_hbm,
               idxa, idxb, bufa, bufb, sema, semb):
    wid = lax.axis_index("s") * NC + lax.axis_index("c")

    @pl.loop(0, CH_PER_TILE)
    def _(j):
        chunk = j * NW + wid

        @pl.when(chunk < NCHUNKS)
        def _():
            base = chunk * CHUNK
            pltpu.sync_copy(row_hbm.at[pl.ds(base, CHUNK)], idxa)
            pltpu.sync_copy(col_hbm.at[pl.ds(base, CHUNK)], idxb)
            ca = pltpu.async_copy(h_hbm.at[idxa], bufa, sema)
            cb = pltpu.async_copy(h_hbm.at[idxb], bufb, semb)
            ca.wait()
            cb.wait()
            pltpu.sync_copy(bufa, ga_hbm.at[pl.ds(base, CHUNK)])
            pltpu.sync_copy(bufb, gb_hbm.at[pl.ds(base, CHUNK)])


_BLK = 2000


def _mlp_body(ga, gb, ea, cd, em, w1a, w1b, w1c, b1, w2, b2, w3, out):
    x = jnp.dot(ga[...], w1a[...], preferred_element_type=jnp.float32)
    x = x + jnp.dot(gb[...], w1b[...], preferred_element_type=jnp.float32)
    x = x + jnp.dot(ea[...], w1c[...], preferred_element_type=jnp.float32)
    x = x + b1[...]
    x = x / (1.0 + jnp.exp(-x))
    x = jnp.dot(x, w2[...], preferred_element_type=jnp.float32) + b2[...]
    x = x / (1.0 + jnp.exp(-x))
    m = jnp.sum(x * w3[...], axis=1, keepdims=True)
    out[...] = cd[...] * m * em[...]


def _tc_mlp(ga, gb, edge_attr, cd16, edge_mask, w1a, w1b, w1c, b1, w2, b2, w3):
    grid = (N_EDGES // _BLK,)
    full = lambda shape: pl.BlockSpec(shape, lambda i: (0, 0))
    return pl.pallas_call(
        _mlp_body,
        grid=grid,
        in_specs=[
            pl.BlockSpec((_BLK, HIDDEN), lambda i: (i, 0)),
            pl.BlockSpec((_BLK, HIDDEN), lambda i: (i, 0)),
            pl.BlockSpec((_BLK, EDGES_IN_D), lambda i: (i, 0)),
            pl.BlockSpec((_BLK, PAD), lambda i: (i, 0)),
            pl.BlockSpec((_BLK, 1), lambda i: (i, 0)),
            full((HIDDEN, HIDDEN)),
            full((HIDDEN, HIDDEN)),
            full((EDGES_IN_D, HIDDEN)),
            full((1, HIDDEN)),
            full((HIDDEN, HIDDEN)),
            full((1, HIDDEN)),
            full((1, HIDDEN)),
        ],
        out_specs=pl.BlockSpec((_BLK, PAD), lambda i: (i, 0)),
        out_shape=jax.ShapeDtypeStruct((N_EDGES, PAD), jnp.float32),
    )(ga, gb, edge_attr, cd16, edge_mask, w1a, w1b, w1c, b1, w2, b2, w3)


@functools.partial(
    pl.kernel,
    out_type=jax.ShapeDtypeStruct((NC, N_NODES, PAD), jnp.float32),
    mesh=_SC_MESH,
    scratch_types=[
        pltpu.VMEM((CHUNK,), jnp.int32),
        pltpu.VMEM((CHUNK, PAD), jnp.float32),
        pltpu.VMEM((ROWS_PER_TILE, PAD), jnp.float32),
        pltpu.VMEM_SHARED((N_NODES, PAD), jnp.float32),
    ],
)
def _sc_scatter(tp_hbm, row_hbm, agg_hbm, idx, buf, zbuf, shared):
    cid = lax.axis_index("c")
    sid = lax.axis_index("s")

    @pl.loop(0, ROWS_PER_TILE)
    def _(i):
        zbuf[i] = jnp.zeros((PAD,), jnp.float32)

    pltpu.sync_copy(zbuf, shared.at[pl.ds(sid * ROWS_PER_TILE, ROWS_PER_TILE)])
    plsc.subcore_barrier()

    @pl.loop(0, SC_CH_PER_TILE)
    def _(j):
        chunk = j * NS + sid

        @pl.when(chunk < SC_NCHUNKS)
        def _():
            base = cid * (N_EDGES // NC) + chunk * CHUNK
            pltpu.sync_copy(row_hbm.at[pl.ds(base, CHUNK)], idx)
            pltpu.sync_copy(tp_hbm.at[pl.ds(base, CHUNK)], buf)
            pltpu.sync_copy(buf, shared.at[idx], add=True)

    plsc.subcore_barrier()
    pltpu.sync_copy(
        shared.at[pl.ds(sid * ROWS_PER_TILE, ROWS_PER_TILE)],
        agg_hbm.at[cid].at[pl.ds(sid * ROWS_PER_TILE, ROWS_PER_TILE)],
    )


def _fin_body(coord, agg, nm, out):
    s = agg[0] + agg[1]
    out[...] = (coord[...] + s[:, :3] * (1.0 / NORM)) * nm[...]


def _tc_fin(coord, agg, node_mask):
    return pl.pallas_call(
        _fin_body,
        out_shape=jax.ShapeDtypeStruct((N_NODES, 3), jnp.float32),
    )(coord, agg, node_mask)


def kernel(h, coord, edge_index, coord_diff, edge_attr, node_mask, edge_mask,
           W1, b1, W2, b2, W3):
    row = edge_index[0].astype(jnp.int32)
    col = edge_index[1].astype(jnp.int32)

    ga, gb = _sc_gather(h, row, col)

    cd16 = jnp.concatenate(
        [coord_diff, jnp.zeros((N_EDGES, PAD - 3), jnp.float32)], axis=1)
    w1a = W1[:, :HIDDEN].T
    w1b = W1[:, HIDDEN:2 * HIDDEN].T
    w1c = W1[:, 2 * HIDDEN:].T
    tp = _tc_mlp(ga, gb, edge_attr, cd16, edge_mask,
                 w1a, w1b, w1c, b1.reshape(1, -1), W2.T, b2.reshape(1, -1),
                 W3.reshape(1, -1))

    agg = _sc_scatter(tp, row)
    return _tc_fin(coord, agg, node_mask)


# speedup vs baseline: 2.8885x; 2.8885x over previous
import functools

import jax
import jax.numpy as jnp
from jax import lax
from jax.experimental import pallas as pl
from jax.experimental.pallas import tpu as pltpu
from jax.experimental.pallas import tpu_sc as plsc

N_NODES = 10000
HIDDEN = 128
N_EDGES = 320000
EDGES_IN_D = 4
NORM = 100.0

NC = 2
NS = 16
NW = NC * NS
CHUNK = 128
NCHUNKS = N_EDGES // CHUNK
CH_PER_TILE = -(-NCHUNKS // NW)
SC_NCHUNKS = NCHUNKS // NC
SC_CH_PER_TILE = -(-SC_NCHUNKS // NS)
N_NODES_PAD = 10240
ROWS_PER_TILE = N_NODES_PAD // NS
PAD = 16

_SC_MESH = plsc.VectorSubcoreMesh(core_axis_name="c", subcore_axis_name="s")


@functools.partial(
    pl.kernel,
    out_type=[
        jax.ShapeDtypeStruct((N_EDGES, HIDDEN), jnp.float32),
        jax.ShapeDtypeStruct((N_EDGES, HIDDEN), jnp.float32),
    ],
    mesh=_SC_MESH,
    scratch_types=[
        pltpu.VMEM((CHUNK,), jnp.int32),
        pltpu.VMEM((CHUNK,), jnp.int32),
        pltpu.VMEM((CHUNK, HIDDEN), jnp.float32),
        pltpu.VMEM((CHUNK, HIDDEN), jnp.float32),
        pltpu.SemaphoreType.DMA,
        pltpu.SemaphoreType.DMA,
    ],
)
def _sc_gather(h_hbm, row_hbm, col_hbm, ga_hbm, gb_hbm,
               idxa, idxb, bufa, bufb, sema, semb):
    wid = lax.axis_index("s") * NC + lax.axis_index("c")

    @pl.loop(0, CH_PER_TILE)
    def _(j):
        chunk = j * NW + wid

        @pl.when(chunk < NCHUNKS)
        def _():
            base = chunk * CHUNK
            pltpu.sync_copy(row_hbm.at[pl.ds(base, CHUNK)], idxa)
            pltpu.sync_copy(col_hbm.at[pl.ds(base, CHUNK)], idxb)
            ca = pltpu.async_copy(h_hbm.at[idxa], bufa, sema)
            cb = pltpu.async_copy(h_hbm.at[idxb], bufb, semb)
            ca.wait()
            cb.wait()
            pltpu.sync_copy(bufa, ga_hbm.at[pl.ds(base, CHUNK)])
            pltpu.sync_copy(bufb, gb_hbm.at[pl.ds(base, CHUNK)])


_BLK = 2000


def _mlp_body(ga, gb, ea, cd, em, w1a, w1b, w1c, b1, w2, b2, w3, out):
    x = jnp.dot(ga[...], w1a[...], preferred_element_type=jnp.float32)
    x = x + jnp.dot(gb[...], w1b[...], preferred_element_type=jnp.float32)
    x = x + jnp.dot(ea[...], w1c[...], preferred_element_type=jnp.float32)
    x = x + b1[...]
    x = x / (1.0 + jnp.exp(-x))
    x = jnp.dot(x, w2[...], preferred_element_type=jnp.float32) + b2[...]
    x = x / (1.0 + jnp.exp(-x))
    m = jnp.sum(x * w3[...], axis=1, keepdims=True)
    out[...] = cd[...] * m * em[...]


def _tc_mlp(ga, gb, edge_attr, cd16, edge_mask, w1a, w1b, w1c, b1, w2, b2, w3):
    grid = (N_EDGES // _BLK,)
    full = lambda shape: pl.BlockSpec(shape, lambda i: (0, 0))
    return pl.pallas_call(
        _mlp_body,
        grid=grid,
        in_specs=[
            pl.BlockSpec((_BLK, HIDDEN), lambda i: (i, 0)),
            pl.BlockSpec((_BLK, HIDDEN), lambda i: (i, 0)),
            pl.BlockSpec((_BLK, EDGES_IN_D), lambda i: (i, 0)),
            pl.BlockSpec((_BLK, PAD), lambda i: (i, 0)),
            pl.BlockSpec((_BLK, 1), lambda i: (i, 0)),
            full((HIDDEN, HIDDEN)),
            full((HIDDEN, HIDDEN)),
            full((EDGES_IN_D, HIDDEN)),
            full((1, HIDDEN)),
            full((HIDDEN, HIDDEN)),
            full((1, HIDDEN)),
            full((1, HIDDEN)),
        ],
        out_specs=pl.BlockSpec((_BLK, PAD), lambda i: (i, 0)),
        out_shape=jax.ShapeDtypeStruct((N_EDGES, PAD), jnp.float32),
    )(ga, gb, edge_attr, cd16, edge_mask, w1a, w1b, w1c, b1, w2, b2, w3)


@functools.partial(
    pl.kernel,
    out_type=[
        jax.ShapeDtypeStruct((N_NODES_PAD, PAD), jnp.float32),
        jax.ShapeDtypeStruct((N_NODES_PAD, PAD), jnp.float32),
    ],
    mesh=_SC_MESH,
    scratch_types=[
        pltpu.VMEM((CHUNK,), jnp.int32),
        pltpu.VMEM((CHUNK, PAD), jnp.float32),
        pltpu.VMEM_SHARED((N_NODES_PAD, PAD), jnp.float32),
    ],
)
def _sc_scatter(tp_hbm, row_hbm, zeros_hbm, agg0_hbm, agg1_hbm,
                idx, buf, shared):
    cid = lax.axis_index("c")
    sid = lax.axis_index("s")
    rbase = sid * ROWS_PER_TILE

    pltpu.sync_copy(zeros_hbm.at[pl.ds(rbase, ROWS_PER_TILE)],
                    shared.at[pl.ds(rbase, ROWS_PER_TILE)])
    plsc.subcore_barrier()

    @pl.loop(0, SC_CH_PER_TILE)
    def _(j):
        chunk = j * NS + sid

        @pl.when(chunk < SC_NCHUNKS)
        def _():
            base = cid * (N_EDGES // NC) + chunk * CHUNK
            pltpu.sync_copy(row_hbm.at[pl.ds(base, CHUNK)], idx)
            pltpu.sync_copy(tp_hbm.at[pl.ds(base, CHUNK)], buf)
            pltpu.sync_copy(buf, shared.at[idx], add=True)

    plsc.subcore_barrier()

    @pl.when(cid == 0)
    def _():
        pltpu.sync_copy(shared.at[pl.ds(rbase, ROWS_PER_TILE)],
                        agg0_hbm.at[pl.ds(rbase, ROWS_PER_TILE)])

    @pl.when(cid == 1)
    def _():
        pltpu.sync_copy(shared.at[pl.ds(rbase, ROWS_PER_TILE)],
                        agg1_hbm.at[pl.ds(rbase, ROWS_PER_TILE)])


def _fin_body(coord, agg0, agg1, nm, out):
    s = agg0[...] + agg1[...]
    out[...] = (coord[...] + s[:N_NODES, :3] * (1.0 / NORM)) * nm[...]


def _tc_fin(coord, agg0, agg1, node_mask):
    return pl.pallas_call(
        _fin_body,
        out_shape=jax.ShapeDtypeStruct((N_NODES, 3), jnp.float32),
    )(coord, agg0, agg1, node_mask)


def kernel(h, coord, edge_index, coord_diff, edge_attr, node_mask, edge_mask,
           W1, b1, W2, b2, W3):
    row = edge_index[0].astype(jnp.int32)
    col = edge_index[1].astype(jnp.int32)

    ga, gb = _sc_gather(h, row, col)

    cd16 = jnp.concatenate(
        [coord_diff, jnp.zeros((N_EDGES, PAD - 3), jnp.float32)], axis=1)
    w1a = W1[:, :HIDDEN].T
    w1b = W1[:, HIDDEN:2 * HIDDEN].T
    w1c = W1[:, 2 * HIDDEN:].T
    tp = _tc_mlp(ga, gb, edge_attr, cd16, edge_mask,
                 w1a, w1b, w1c, b1.reshape(1, -1), W2.T, b2.reshape(1, -1),
                 W3.reshape(1, -1))

    zeros = jnp.zeros((N_NODES_PAD, PAD), jnp.float32)
    agg0, agg1 = _sc_scatter(tp, row, zeros)
    return _tc_fin(coord, agg0, agg1, node_mask)
